# baseline (speedup 1.0000x reference)
import jax
import jax.numpy as jnp
from jax import lax
from jax.experimental import pallas as pl
from jax.experimental.pallas import tpu as pltpu
from jax.experimental.pallas import tpu_sc as plsc

B = 16384
D = 32
NC = 2
NS = 16
NW = NC * NS
BPW = B // NW
NT = 4


def _emb_body(cat_i, sub_i, ele_i, brd_i,
              merged_t, ele_t,
              out,
              idx0, idx1, idx2, idx3,
              rows0, rows1, rows2, rows3,
              sem_i, sem_g, sem_o):
    wid = lax.axis_index("s") * NC + lax.axis_index("c")
    base = wid * BPW
    idxs = (idx0, idx1, idx2, idx3)
    rows = (rows0, rows1, rows2, rows3)
    tabs = (merged_t, merged_t, ele_t, merged_t)

    ics = [
        pltpu.async_copy(s.at[pl.ds(base, BPW)], idxs[t], sem_i)
        for t, s in enumerate((cat_i, sub_i, ele_i, brd_i))
    ]
    for c in ics:
        c.wait()

    gcs = [
        pltpu.async_copy(tabs[t].at[idxs[t]], rows[t], sem_g)
        for t in range(NT)
    ]
    ocs = []
    for t in range(NT):
        gcs[t].wait()
        ocs.append(pltpu.async_copy(
            rows[t], out.at[pl.ds(t * B + base, BPW)], sem_o))
    for c in ocs:
        c.wait()


def kernel(categories, sub_categories, elements, brands,
           category_table, sub_category_table, element_table, brand_table):
    n_cat = category_table.shape[0]
    n_sub = sub_category_table.shape[0]
    merged = jnp.concatenate(
        [category_table, sub_category_table, brand_table], axis=0)
    sub_categories = sub_categories + n_cat
    brands = brands + (n_cat + n_sub)
    mesh = plsc.VectorSubcoreMesh(core_axis_name="c", subcore_axis_name="s")
    f = pl.kernel(
        _emb_body,
        mesh=mesh,
        compiler_params=pltpu.CompilerParams(use_tc_tiling_on_sc=False),
        out_type=jax.ShapeDtypeStruct((NT * B, D), jnp.float32),
        scratch_types=(
            [pltpu.VMEM((BPW,), jnp.int32) for _ in range(NT)]
            + [pltpu.VMEM((BPW, D), jnp.float32) for _ in range(NT)]
            + [pltpu.SemaphoreType.DMA,
               pltpu.SemaphoreType.DMA,
               pltpu.SemaphoreType.DMA]
        ),
    )
    return f(categories, sub_categories, elements, brands,
             merged, element_table)


# speedup vs baseline: 1.1271x; 1.1271x over previous
import jax
import jax.numpy as jnp
from jax import lax
from jax.experimental import pallas as pl
from jax.experimental.pallas import tpu as pltpu
from jax.experimental.pallas import tpu_sc as plsc

B = 16384
D = 32
NC = 2
NS = 16
NW = NC * NS
BPW = B // NW
NT = 4


def _emb_body(cat_i, sub_i, ele_i, brd_i,
              cat_t, sub_t, ele_t, brd_t,
              out,
              idx0, idx1, idx2, idx3,
              rows0, rows1, rows2, rows3,
              sem_i, sem_g, sem_o):
    wid = lax.axis_index("s") * NC + lax.axis_index("c")
    base = wid * BPW
    idxs = (idx0, idx1, idx2, idx3)
    rows = (rows0, rows1, rows2, rows3)
    tabs = (cat_t, sub_t, ele_t, brd_t)

    ics = [
        pltpu.async_copy(s.at[pl.ds(base, BPW)], idxs[t], sem_i)
        for t, s in enumerate((cat_i, sub_i, ele_i, brd_i))
    ]
    for c in ics:
        c.wait()

    gcs = [
        pltpu.async_copy(tabs[t].at[idxs[t]], rows[t], sem_g)
        for t in range(NT)
    ]
    ocs = []
    for t in range(NT):
        gcs[t].wait()
        ocs.append(pltpu.async_copy(
            rows[t], out.at[pl.ds(t * B + base, BPW)], sem_o))
    for c in ocs:
        c.wait()


def kernel(categories, sub_categories, elements, brands,
           category_table, sub_category_table, element_table, brand_table):
    mesh = plsc.VectorSubcoreMesh(core_axis_name="c", subcore_axis_name="s")
    f = pl.kernel(
        _emb_body,
        mesh=mesh,
        compiler_params=pltpu.CompilerParams(use_tc_tiling_on_sc=False),
        out_type=jax.ShapeDtypeStruct((NT * B, D), jnp.float32),
        scratch_types=(
            [pltpu.VMEM((BPW,), jnp.int32) for _ in range(NT)]
            + [pltpu.VMEM((BPW, D), jnp.float32) for _ in range(NT)]
            + [pltpu.SemaphoreType.DMA,
               pltpu.SemaphoreType.DMA,
               pltpu.SemaphoreType.DMA]
        ),
    )
    return f(categories, sub_categories, elements, brands,
             category_table, sub_category_table, element_table, brand_table)
